# baseline (device time: 23730 ns/iter reference)
import jax
import jax.numpy as jnp
from jax import lax
from jax.experimental import pallas as pl
from jax.experimental.pallas import tpu as pltpu

N_DEV = 8
B, SQ, DM, HQ_TOT, DH = 2, 256, 512, 32, 64
H_PER = HQ_TOT // N_DEV
BLK = 64
ROWS = B * SQ
CHUNK = ROWS // N_DEV


def _body(x_ref, wq_ref, k_ref, v_ref, wo_ref, out_ref,
          ctx_ref, part_ref, p1_ref, red_ref,
          p1_send, p1_recv, p2_send, p2_recv):
    my = lax.axis_index("i")

    q = jnp.dot(x_ref[...], wq_ref[...], preferred_element_type=jnp.float32)
    q = (q * 0.125).astype(jnp.bfloat16)

    r_blk = lax.broadcasted_iota(jnp.int32, (SQ, SQ), 0) // BLK
    c_blk = lax.broadcasted_iota(jnp.int32, (SQ, SQ), 1) // BLK
    mask = r_blk == c_blk

    for b in range(B):
        rows = slice(b * SQ, (b + 1) * SQ)
        for h in range(H_PER):
            cols = slice(h * DH, (h + 1) * DH)
            qh = q[rows, cols]
            scores = lax.dot_general(
                qh, k_ref[b, :, h, :], (((1,), (1,)), ((), ())),
                preferred_element_type=jnp.float32)
            scores = jnp.where(mask, scores, -1e9)
            m = jnp.max(scores, axis=1, keepdims=True)
            w = jnp.exp(scores - m)
            w = (w / jnp.sum(w, axis=1, keepdims=True)).astype(jnp.bfloat16)
            ctx = jnp.dot(w, v_ref[b, :, h, :],
                          preferred_element_type=jnp.float32)
            ctx_ref[rows, cols] = ctx.astype(jnp.bfloat16)

    barrier = pltpu.get_barrier_semaphore()
    for k in range(1, N_DEV):
        pl.semaphore_signal(barrier, inc=1,
                            device_id=(lax.rem(my + k, N_DEV),),
                            device_id_type=pltpu.DeviceIdType.MESH)
    pl.semaphore_wait(barrier, N_DEV - 1)

    p1 = []
    for k in range(1, N_DEV):
        d = lax.rem(my + k, N_DEV)
        rows = pl.ds(d * CHUNK, CHUNK)
        part_ref[rows, :] = jnp.dot(
            ctx_ref[rows, :], wo_ref[...],
            preferred_element_type=jnp.float32).astype(jnp.bfloat16)
        rdma = pltpu.make_async_remote_copy(
            src_ref=part_ref.at[rows, :],
            dst_ref=p1_ref.at[k - 1],
            send_sem=p1_send.at[k - 1],
            recv_sem=p1_recv.at[k - 1],
            device_id=(d,),
            device_id_type=pltpu.DeviceIdType.MESH,
        )
        rdma.start()
        p1.append(rdma)

    acc = jnp.dot(ctx_ref[pl.ds(my * CHUNK, CHUNK), :], wo_ref[...],
                  preferred_element_type=jnp.float32)
    for j in range(N_DEV - 1):
        p1[j].wait_recv()
        acc = acc + p1_ref[j].astype(jnp.float32)
    red_ref[...] = acc.astype(jnp.bfloat16)
    out_ref[pl.ds(my * CHUNK, CHUNK), :] = red_ref[...]

    p2 = []
    for k in range(1, N_DEV):
        d = lax.rem(my + k, N_DEV)
        rdma = pltpu.make_async_remote_copy(
            src_ref=red_ref,
            dst_ref=out_ref.at[pl.ds(my * CHUNK, CHUNK), :],
            send_sem=p2_send.at[k - 1],
            recv_sem=p2_recv.at[k - 1],
            device_id=(d,),
            device_id_type=pltpu.DeviceIdType.MESH,
        )
        rdma.start()
        p2.append(rdma)

    for j in range(N_DEV - 1):
        p1[j].wait_send()

    for j in range(N_DEV - 1):
        sdev = lax.rem(my + N_DEV - (j + 1), N_DEV)
        recv = pltpu.make_async_remote_copy(
            src_ref=red_ref,
            dst_ref=out_ref.at[pl.ds(sdev * CHUNK, CHUNK), :],
            send_sem=p2_send.at[j],
            recv_sem=p2_recv.at[j],
            device_id=(sdev,),
            device_id_type=pltpu.DeviceIdType.MESH,
        )
        recv.wait_recv()

    for j in range(N_DEV - 1):
        p2[j].wait_send()


def kernel(x, Wq, K_ext, V_ext, Wo):
    i = lax.axis_index("i")
    xb = x.reshape(ROWS, DM).astype(jnp.bfloat16)
    wq = Wq.astype(jnp.bfloat16)
    wo = Wo.astype(jnp.bfloat16)
    zero = jnp.zeros((), jnp.int32)
    k_my = lax.dynamic_slice(K_ext, (zero, zero, i * H_PER, zero),
                             (B, SQ, H_PER, DH)).astype(jnp.bfloat16)
    v_my = lax.dynamic_slice(V_ext, (zero, zero, i * H_PER, zero),
                             (B, SQ, H_PER, DH)).astype(jnp.bfloat16)

    out = pl.pallas_call(
        _body,
        out_shape=jax.ShapeDtypeStruct((ROWS, DM), jnp.bfloat16),
        in_specs=[pl.BlockSpec(memory_space=pltpu.VMEM)] * 5,
        out_specs=pl.BlockSpec(memory_space=pltpu.VMEM),
        scratch_shapes=[
            pltpu.VMEM((ROWS, H_PER * DH), jnp.bfloat16),
            pltpu.VMEM((ROWS, DM), jnp.bfloat16),
            pltpu.VMEM((N_DEV - 1, CHUNK, DM), jnp.bfloat16),
            pltpu.VMEM((CHUNK, DM), jnp.bfloat16),
            pltpu.SemaphoreType.DMA((N_DEV - 1,)),
            pltpu.SemaphoreType.DMA((N_DEV - 1,)),
            pltpu.SemaphoreType.DMA((N_DEV - 1,)),
            pltpu.SemaphoreType.DMA((N_DEV - 1,)),
        ],
        compiler_params=pltpu.CompilerParams(collective_id=0),
    )(xb, wq, k_my, v_my, wo)
    return out.reshape(B, SQ, DM)


# device time: 21752 ns/iter; 1.0909x vs baseline; 1.0909x over previous
import jax
import jax.numpy as jnp
from jax import lax
from jax.experimental import pallas as pl
from jax.experimental.pallas import tpu as pltpu

N_DEV = 8
B, SQ, DM, HQ_TOT, DH = 2, 256, 512, 32, 64
H_PER = HQ_TOT // N_DEV
BLK = 64
ROWS = B * SQ
CHUNK = ROWS // N_DEV
HALF = CHUNK // 2


def _body(x_ref, wq_ref, k_ref, v_ref, wo_ref, out_ref,
          ctx_ref, part_ref, p1_ref, red_ref,
          p1_send, p1_recv, p2_send, p2_recv):
    my = lax.axis_index("i")

    q = jnp.dot(x_ref[...], wq_ref[...], preferred_element_type=jnp.float32)
    q = (q * 0.125).astype(jnp.bfloat16)

    r_blk = lax.broadcasted_iota(jnp.int32, (SQ, SQ), 0) // BLK
    c_blk = lax.broadcasted_iota(jnp.int32, (SQ, SQ), 1) // BLK
    mask = r_blk == c_blk

    for b in range(B):
        rows = slice(b * SQ, (b + 1) * SQ)
        for h in range(H_PER):
            cols = slice(h * DH, (h + 1) * DH)
            qh = q[rows, cols]
            scores = lax.dot_general(
                qh, k_ref[h, b], (((1,), (1,)), ((), ())),
                preferred_element_type=jnp.float32)
            scores = jnp.where(mask, scores, -1e9)
            m = jnp.max(scores, axis=1, keepdims=True)
            w = jnp.exp(scores - m)
            w = (w / jnp.sum(w, axis=1, keepdims=True)).astype(jnp.bfloat16)
            ctx = jnp.dot(w, v_ref[h, b],
                          preferred_element_type=jnp.float32)
            ctx_ref[rows, cols] = ctx.astype(jnp.bfloat16)

    barrier = pltpu.get_barrier_semaphore()
    for k in range(1, N_DEV):
        pl.semaphore_signal(barrier, inc=1,
                            device_id=(lax.rem(my + k, N_DEV),),
                            device_id_type=pltpu.DeviceIdType.MESH)
    pl.semaphore_wait(barrier, N_DEV - 1)

    p1 = []
    for k in range(1, N_DEV):
        d = lax.rem(my + k, N_DEV)
        rows = pl.ds(d * CHUNK, CHUNK)
        part_ref[rows, :] = jnp.dot(
            ctx_ref[rows, :], wo_ref[...],
            preferred_element_type=jnp.float32).astype(jnp.bfloat16)
        rdma = pltpu.make_async_remote_copy(
            src_ref=part_ref.at[pl.ds(d * CHUNK, HALF), :],
            dst_ref=p1_ref.at[0, k - 1],
            send_sem=p1_send.at[0, k - 1],
            recv_sem=p1_recv.at[0, k - 1],
            device_id=(d,),
            device_id_type=pltpu.DeviceIdType.MESH,
        )
        rdma.start()
        p1.append(rdma)
    for k in range(1, N_DEV):
        d = lax.rem(my + k, N_DEV)
        rdma = pltpu.make_async_remote_copy(
            src_ref=part_ref.at[pl.ds(d * CHUNK + HALF, HALF), :],
            dst_ref=p1_ref.at[1, k - 1],
            send_sem=p1_send.at[1, k - 1],
            recv_sem=p1_recv.at[1, k - 1],
            device_id=(d,),
            device_id_type=pltpu.DeviceIdType.MESH,
        )
        rdma.start()
        p1.append(rdma)

    own = jnp.dot(ctx_ref[pl.ds(my * CHUNK, CHUNK), :], wo_ref[...],
                  preferred_element_type=jnp.float32)

    p2 = []
    for g in range(2):
        acc = own[g * HALF:(g + 1) * HALF, :]
        for j in range(N_DEV - 1):
            p1[g * (N_DEV - 1) + j].wait_recv()
            acc = acc + p1_ref[g, j].astype(jnp.float32)
        red_ref[pl.ds(g * HALF, HALF), :] = acc.astype(jnp.bfloat16)
        myrows = pl.ds(my * CHUNK + g * HALF, HALF)
        out_ref[myrows, :] = red_ref[pl.ds(g * HALF, HALF), :]
        for k in range(1, N_DEV):
            d = lax.rem(my + k, N_DEV)
            rdma = pltpu.make_async_remote_copy(
                src_ref=red_ref.at[pl.ds(g * HALF, HALF), :],
                dst_ref=out_ref.at[myrows, :],
                send_sem=p2_send.at[g, k - 1],
                recv_sem=p2_recv.at[g, k - 1],
                device_id=(d,),
                device_id_type=pltpu.DeviceIdType.MESH,
            )
            rdma.start()
            p2.append(rdma)

    for r in p1:
        r.wait_send()

    for g in range(2):
        for j in range(N_DEV - 1):
            sdev = lax.rem(my + N_DEV - (j + 1), N_DEV)
            recv = pltpu.make_async_remote_copy(
                src_ref=red_ref.at[pl.ds(g * HALF, HALF), :],
                dst_ref=out_ref.at[pl.ds(sdev * CHUNK + g * HALF, HALF), :],
                send_sem=p2_send.at[g, j],
                recv_sem=p2_recv.at[g, j],
                device_id=(sdev,),
                device_id_type=pltpu.DeviceIdType.MESH,
            )
            recv.wait_recv()

    for r in p2:
        r.wait_send()


def kernel(x, Wq, K_ext, V_ext, Wo):
    i = lax.axis_index("i")
    xb = x.reshape(ROWS, DM).astype(jnp.bfloat16)
    wq = Wq.astype(jnp.bfloat16)
    wo = Wo.astype(jnp.bfloat16)
    zero = jnp.zeros((), jnp.int32)
    k_s = lax.dynamic_slice(K_ext, (zero, zero, i * H_PER, zero),
                            (B, SQ, H_PER, DH)).astype(jnp.bfloat16)
    v_s = lax.dynamic_slice(V_ext, (zero, zero, i * H_PER, zero),
                            (B, SQ, H_PER, DH)).astype(jnp.bfloat16)
    k_my = jnp.transpose(k_s, (2, 0, 1, 3))
    v_my = jnp.transpose(v_s, (2, 0, 1, 3))

    out = pl.pallas_call(
        _body,
        out_shape=jax.ShapeDtypeStruct((ROWS, DM), jnp.bfloat16),
        in_specs=[pl.BlockSpec(memory_space=pltpu.VMEM)] * 5,
        out_specs=pl.BlockSpec(memory_space=pltpu.VMEM),
        scratch_shapes=[
            pltpu.VMEM((ROWS, H_PER * DH), jnp.bfloat16),
            pltpu.VMEM((ROWS, DM), jnp.bfloat16),
            pltpu.VMEM((2, N_DEV - 1, HALF, DM), jnp.bfloat16),
            pltpu.VMEM((CHUNK, DM), jnp.bfloat16),
            pltpu.SemaphoreType.DMA((2, N_DEV - 1)),
            pltpu.SemaphoreType.DMA((2, N_DEV - 1)),
            pltpu.SemaphoreType.DMA((2, N_DEV - 1)),
            pltpu.SemaphoreType.DMA((2, N_DEV - 1)),
        ],
        compiler_params=pltpu.CompilerParams(collective_id=0),
    )(xb, wq, k_my, v_my, wo)
    return out.reshape(B, SQ, DM)


# device time: 21649 ns/iter; 1.0961x vs baseline; 1.0048x over previous
import jax
import jax.numpy as jnp
from jax import lax
from jax.experimental import pallas as pl
from jax.experimental.pallas import tpu as pltpu

N_DEV = 8
B, SQ, DM, HQ_TOT, DH = 2, 256, 512, 32, 64
H_PER = HQ_TOT // N_DEV
BLK = 64
ROWS = B * SQ
CHUNK = ROWS // N_DEV
HALF = CHUNK // 2


def _body(x_ref, wq_ref, k_ref, v_ref, wo_ref, out_ref,
          ctx_ref, part_ref, p1_ref, red_ref,
          p1_send, p1_recv, p2_send, p2_recv):
    my = lax.axis_index("i")

    q = jnp.dot(x_ref[...], wq_ref[...], preferred_element_type=jnp.float32)
    q = (q * 0.125).astype(jnp.bfloat16)

    r_blk = lax.broadcasted_iota(jnp.int32, (SQ, SQ), 0) // BLK
    c_blk = lax.broadcasted_iota(jnp.int32, (SQ, SQ), 1) // BLK
    mask = r_blk == c_blk

    for b in range(B):
        rows = slice(b * SQ, (b + 1) * SQ)
        for h in range(H_PER):
            cols = slice(h * DH, (h + 1) * DH)
            qh = q[rows, cols]
            scores = lax.dot_general(
                qh, k_ref[h, b], (((1,), (1,)), ((), ())),
                preferred_element_type=jnp.float32)
            scores = jnp.where(mask, scores, -1e9)
            m = jnp.max(scores, axis=1, keepdims=True)
            w = jnp.exp(scores - m)
            w = (w / jnp.sum(w, axis=1, keepdims=True)).astype(jnp.bfloat16)
            ctx = jnp.dot(w, v_ref[h, b],
                          preferred_element_type=jnp.float32)
            ctx_ref[rows, cols] = ctx.astype(jnp.bfloat16)

    barrier = pltpu.get_barrier_semaphore()
    for k in range(1, N_DEV):
        pl.semaphore_signal(barrier, inc=1,
                            device_id=(lax.rem(my + k, N_DEV),),
                            device_id_type=pltpu.DeviceIdType.MESH)
    pl.semaphore_wait(barrier, N_DEV - 1)

    p1 = []
    for k in range(1, N_DEV):
        d = lax.rem(my + k, N_DEV)
        rows = pl.ds(d * CHUNK, CHUNK)
        part_ref[rows, :] = jnp.dot(
            ctx_ref[rows, :], wo_ref[...],
            preferred_element_type=jnp.float32).astype(jnp.bfloat16)
        rdma = pltpu.make_async_remote_copy(
            src_ref=part_ref.at[pl.ds(d * CHUNK, HALF), :],
            dst_ref=p1_ref.at[0, k - 1],
            send_sem=p1_send.at[0, k - 1],
            recv_sem=p1_recv.at[0, k - 1],
            device_id=(d,),
            device_id_type=pltpu.DeviceIdType.MESH,
        )
        rdma.start()
        p1.append(rdma)
    for k in range(1, N_DEV):
        d = lax.rem(my + k, N_DEV)
        rdma = pltpu.make_async_remote_copy(
            src_ref=part_ref.at[pl.ds(d * CHUNK + HALF, HALF), :],
            dst_ref=p1_ref.at[1, k - 1],
            send_sem=p1_send.at[1, k - 1],
            recv_sem=p1_recv.at[1, k - 1],
            device_id=(d,),
            device_id_type=pltpu.DeviceIdType.MESH,
        )
        rdma.start()
        p1.append(rdma)

    own = jnp.dot(ctx_ref[pl.ds(my * CHUNK, CHUNK), :], wo_ref[...],
                  preferred_element_type=jnp.float32)

    p2 = []
    for g in range(2):
        acc = own[g * HALF:(g + 1) * HALF, :]
        for j in range(N_DEV - 1):
            p1[g * (N_DEV - 1) + j].wait_recv()
            acc = acc + p1_ref[g, j].astype(jnp.float32)
        red_ref[pl.ds(g * HALF, HALF), :] = acc.astype(jnp.bfloat16)
        myrows = pl.ds(my * CHUNK + g * HALF, HALF)
        out_ref[myrows, :] = red_ref[pl.ds(g * HALF, HALF), :]
        for k in range(1, N_DEV):
            d = lax.rem(my + k, N_DEV)
            rdma = pltpu.make_async_remote_copy(
                src_ref=red_ref.at[pl.ds(g * HALF, HALF), :],
                dst_ref=out_ref.at[myrows, :],
                send_sem=p2_send.at[g, k - 1],
                recv_sem=p2_recv.at[g, k - 1],
                device_id=(d,),
                device_id_type=pltpu.DeviceIdType.MESH,
            )
            rdma.start()
            p2.append(rdma)

    for r in p1:
        r.wait_send()

    for g in range(2):
        for j in range(N_DEV - 1):
            sdev = lax.rem(my + N_DEV - (j + 1), N_DEV)
            recv = pltpu.make_async_remote_copy(
                src_ref=red_ref.at[pl.ds(g * HALF, HALF), :],
                dst_ref=out_ref.at[pl.ds(sdev * CHUNK + g * HALF, HALF), :],
                send_sem=p2_send.at[g, j],
                recv_sem=p2_recv.at[g, j],
                device_id=(sdev,),
                device_id_type=pltpu.DeviceIdType.MESH,
            )
            recv.wait_recv()

    for r in p2:
        r.wait_send()


def kernel(x, Wq, K_ext, V_ext, Wo):
    i = lax.axis_index("i")
    xb = x.reshape(ROWS, DM).astype(jnp.bfloat16)
    wq = Wq.astype(jnp.bfloat16)
    wo = Wo.astype(jnp.bfloat16)
    zero = jnp.zeros((), jnp.int32)
    k_s = lax.dynamic_slice(K_ext, (zero, zero, i * H_PER, zero),
                            (B, SQ, H_PER, DH))
    v_s = lax.dynamic_slice(V_ext, (zero, zero, i * H_PER, zero),
                            (B, SQ, H_PER, DH))
    k_my = jnp.transpose(k_s, (2, 0, 1, 3)).astype(jnp.bfloat16)
    v_my = jnp.transpose(v_s, (2, 0, 1, 3)).astype(jnp.bfloat16)

    out = pl.pallas_call(
        _body,
        out_shape=jax.ShapeDtypeStruct((ROWS, DM), jnp.bfloat16),
        in_specs=[pl.BlockSpec(memory_space=pltpu.VMEM)] * 5,
        out_specs=pl.BlockSpec(memory_space=pltpu.VMEM),
        scratch_shapes=[
            pltpu.VMEM((ROWS, H_PER * DH), jnp.bfloat16),
            pltpu.VMEM((ROWS, DM), jnp.bfloat16),
            pltpu.VMEM((2, N_DEV - 1, HALF, DM), jnp.bfloat16),
            pltpu.VMEM((CHUNK, DM), jnp.bfloat16),
            pltpu.SemaphoreType.DMA((2, N_DEV - 1)),
            pltpu.SemaphoreType.DMA((2, N_DEV - 1)),
            pltpu.SemaphoreType.DMA((2, N_DEV - 1)),
            pltpu.SemaphoreType.DMA((2, N_DEV - 1)),
        ],
        compiler_params=pltpu.CompilerParams(collective_id=0),
    )(xb, wq, k_my, v_my, wo)
    return out.reshape(B, SQ, DM)
